# baseline (speedup 1.0000x reference)
import functools

import jax
import jax.numpy as jnp
from jax import lax
from jax.experimental import pallas as pl
from jax.experimental.pallas import tpu as pltpu
from jax.experimental.pallas import tpu_sc as plsc

D = 128
W = 2 * D + 1
WP = 272
S_DIM, B_DIM, H_DIM = 2048, 32, 1024
CH = 16
NCH = WP // CH
NBUF = 4
LOOKAHEAD = 3
NQ = H_DIM // 16


def _predict_kernel(hid_ref, wp_ref, wpb_ref, vp_ref, vpb_ref,
                    idx_ref, gauss_ref, live_ref):
    h = hid_ref[...]
    wph = lax.dot_general(h, wp_ref[...], (((1,), (1,)), ((), ())),
                          preferred_element_type=jnp.float32)
    wph = jnp.tanh(wph + wpb_ref[...])
    vp8 = jnp.broadcast_to(vp_ref[...], (8, H_DIM))
    z = lax.dot_general(wph, vp8, (((1,), (1,)), ((), ())),
                        preferred_element_type=jnp.float32)[:, :1]
    p = S_DIM * jax.nn.sigmoid(z + vpb_ref[0, 0])
    c = lax.round(p, lax.RoundingMethod.TO_NEAREST_EVEN).astype(jnp.int32)
    j = lax.broadcasted_iota(jnp.int32, (B_DIM, WP), 1)
    b = lax.broadcasted_iota(jnp.int32, (B_DIM, WP), 0)
    s_clamped = jnp.clip(c - D + j, 0, S_DIM - 1)
    idx_ref[...] = s_clamped * B_DIM + b
    j2 = lax.broadcasted_iota(jnp.int32, (B_DIM, WP), 1)
    s_abs = c - D + j2
    live = (s_abs >= 0) & (s_abs < S_DIM) & (j2 < W)
    live_ref[...] = live.astype(jnp.float32)
    wi = s_abs.astype(jnp.float32)
    gauss_ref[...] = jnp.exp((wi - p) ** 2 * (-1.0 / 8192.0))


def _make_sc_attend():
    mesh = plsc.VectorSubcoreMesh(core_axis_name="c", subcore_axis_name="s")
    info = plsc.get_sparse_core_info()
    nc = info.num_cores

    @functools.partial(
        pl.kernel, mesh=mesh,
        out_type=[
            jax.ShapeDtypeStruct((B_DIM * WP,), jnp.float32),
            jax.ShapeDtypeStruct((B_DIM * H_DIM,), jnp.float32),
        ],
        scratch_types=(
            [pltpu.VMEM((WP,), jnp.int32),
             pltpu.VMEM((H_DIM,), jnp.float32),
             pltpu.VMEM((WP,), jnp.float32),
             pltpu.VMEM((WP,), jnp.float32),
             pltpu.VMEM((WP,), jnp.float32),
             pltpu.VMEM((WP,), jnp.float32),
             pltpu.VMEM((WP,), jnp.float32),
             pltpu.VMEM((H_DIM,), jnp.float32)]
            + [pltpu.VMEM((CH, H_DIM), jnp.float32)] * NBUF
            + [pltpu.SemaphoreType.DMA] * NBUF
        ),
    )
    def attend_k(idx_hbm, hid_hbm, gauss_hbm, live_hbm, table_hbm,
                 scaled_hbm, ctx_hbm,
                 idx_v, hv, gauss_v, live_v, sc_v, w_v, so_v, ctx_v, *scr):
        bufs = list(scr[:NBUF])
        gsem = list(scr[NBUF:])
        zero16 = jnp.zeros((16,), jnp.float32)
        wid = lax.axis_index("s") * nc + lax.axis_index("c")
        pltpu.sync_copy(idx_hbm.at[pl.ds(wid * WP, WP)], idx_v)
        pltpu.sync_copy(hid_hbm.at[pl.ds(wid * H_DIM, H_DIM)], hv)
        pltpu.sync_copy(gauss_hbm.at[pl.ds(wid * WP, WP)], gauss_v)
        pltpu.sync_copy(live_hbm.at[pl.ds(wid * WP, WP)], live_v)
        sc_v[pl.ds(WP - 16, 16)] = zero16

        lane = lax.iota(jnp.int32, 16)

        def rnd(x):
            c = x * 65537.0
            return c - (c - x)

        def rnd2(a, b):
            return rnd(a), rnd(b)

        def hstep(q, carry):
            off = pl.ds(q * 16, 16)
            hv[off] = rnd(hv[off])
            return carry
        lax.fori_loop(0, NQ, hstep, 0)

        gd = lax.GatherDimensionNumbers(offset_dims=(), collapsed_slice_dims=(0,),
                                        start_index_map=(0,))

        def _perm(x, idx):
            return lax.gather(x, idx[:, None], gd, slice_sizes=(1,),
                              mode=lax.GatherScatterMode.PROMISE_IN_BOUNDS)

        def _bcast(x, r):
            return _perm(x, jnp.full((16,), r, jnp.int32))

        def _bfly(x, op):
            for sh in (8, 4, 2, 1):
                x = op(x, _perm(x, lane ^ sh))
            return x

        def score_chunk(buf, cj):
            def qstep(q, accs):
                off = pl.ds(q * 16, 16)
                hq = hv[off]
                out = []
                for r in range(0, CH, 2):
                    ra, rb = rnd2(buf[r, off], buf[r + 1, off])
                    out.append(accs[r] + ra * hq)
                    out.append(accs[r + 1] + rb * hq)
                return tuple(out)
            accs = lax.fori_loop(0, NQ, qstep, (zero16,) * CH)
            row_scores = zero16
            for r in range(CH):
                row_scores = jnp.where(lane == r, _bfly(accs[r], jnp.add),
                                       row_scores)
            sc_v[pl.ds(cj * CH, CH)] = row_scores

        def ctx_chunk(buf, cj):
            w16 = rnd(w_v[pl.ds(cj * CH, CH)])
            wbs = [_bcast(w16, r) for r in range(CH)]

            def qstep(q, carry):
                off = pl.ds(q * 16, 16)
                acc = ctx_v[off]
                for r in range(0, CH, 2):
                    ra, rb = rnd2(buf[r, off], buf[r + 1, off])
                    acc = acc + ra * wbs[r]
                    acc = acc + rb * wbs[r + 1]
                ctx_v[off] = acc
                return carry
            lax.fori_loop(0, NQ, qstep, 0)

        def stream_pass(per_chunk):
            gat = [None] * NBUF
            for ci in range(NCH + LOOKAHEAD):
                if ci < NCH:
                    k = ci % NBUF
                    gat[k] = pltpu.async_copy(
                        table_hbm.at[idx_v.at[pl.ds(ci * CH, CH)]],
                        bufs[k], gsem[k])
                cj = ci - LOOKAHEAD
                if 0 <= cj < NCH:
                    kj = cj % NBUF
                    gat[kj].wait()
                    per_chunk(bufs[kj], cj)

        stream_pass(score_chunk)

        mx = zero16
        for cc in range(WP // 16):
            off = pl.ds(cc * 16, 16)
            s = sc_v[off] * live_v[off]
            sc_v[off] = s
            mx = jnp.maximum(mx, s)
        mvec = _bfly(mx, jnp.maximum)

        dacc = zero16
        for cc in range(WP // 16):
            off = pl.ds(cc * 16, 16)
            win = (lax.iota(jnp.int32, 16) + (cc * 16)) < W
            e = jnp.where(win, jnp.exp(sc_v[off] - mvec), zero16)
            sc_v[off] = e
            dacc = dacc + e
        dinv = jnp.full((16,), 1.0) / _bfly(dacc, jnp.add)

        for cc in range(WP // 16):
            off = pl.ds(cc * 16, 16)
            scl = sc_v[off] * dinv * gauss_v[off]
            so_v[off] = scl
            w_v[off] = scl * live_v[off]
        pltpu.sync_copy(so_v, scaled_hbm.at[pl.ds(wid * WP, WP)])

        def zstep(q, carry):
            ctx_v[pl.ds(q * 16, 16)] = zero16
            return carry
        lax.fori_loop(0, NQ, zstep, 0)
        stream_pass(ctx_chunk)
        pltpu.sync_copy(ctx_v, ctx_hbm.at[pl.ds(wid * H_DIM, H_DIM)])

    return attend_k


def kernel(t, hidden, encoder_outputs, Wp_w, Wp_b, vp_w, vp_b):
    S, B, H = encoder_outputs.shape
    idx2, gauss2, live2 = pl.pallas_call(
        _predict_kernel,
        out_shape=(
            jax.ShapeDtypeStruct((B, WP), jnp.int32),
            jax.ShapeDtypeStruct((B, WP), jnp.float32),
            jax.ShapeDtypeStruct((B, WP), jnp.float32),
        ),
        in_specs=[
            pl.BlockSpec((B, H), lambda: (0, 0)),
            pl.BlockSpec((H, H), lambda: (0, 0)),
            pl.BlockSpec((1, H), lambda: (0, 0)),
            pl.BlockSpec((1, H), lambda: (0, 0)),
            pl.BlockSpec(memory_space=pltpu.SMEM),
        ],
        out_specs=(
            pl.BlockSpec((B, WP), lambda: (0, 0)),
            pl.BlockSpec((B, WP), lambda: (0, 0)),
            pl.BlockSpec((B, WP), lambda: (0, 0)),
        ),
    )(hidden, Wp_w, Wp_b.reshape(1, H), vp_w, vp_b.reshape(1, 1))

    table = encoder_outputs.reshape(S * B, H)
    scaled_flat, ctx_flat = _make_sc_attend()(
        idx2.reshape(B * WP), hidden.reshape(B * H),
        gauss2.reshape(B * WP), live2.reshape(B * WP), table)
    return scaled_flat.reshape(B, WP)[:, :W], ctx_flat.reshape(B, H)
